# baseline (speedup 1.0000x reference)
import functools
import jax
import jax.numpy as jnp
from jax import lax
from jax.experimental import pallas as pl
from jax.experimental.pallas import tpu as pltpu
from jax.experimental.pallas import tpu_sc as plsc

_NC = 16
_K = 16


def _sc_gather(table, idx):
    r, d = table.shape
    m = idx.shape[0]
    nw = 32
    bpw = m // nw
    chunk = min(bpw, max(8, (64 * 1024) // (4 * d)))
    nit = bpw // chunk
    mesh = plsc.VectorSubcoreMesh(core_axis_name="c", subcore_axis_name="s")

    @functools.partial(
        pl.kernel, mesh=mesh,
        compiler_params=pltpu.CompilerParams(use_tc_tiling_on_sc=False),
        out_type=jax.ShapeDtypeStruct((m, d), jnp.float32),
        scratch_types=[
            pltpu.VMEM((chunk,), jnp.int32),
            pltpu.VMEM((chunk, d), jnp.float32),
            pltpu.SemaphoreType.DMA,
        ],
    )
    def k(table_hbm, idx_hbm, out_hbm, idx_v, rows_v, sem):
        wid = lax.axis_index("s") * 2 + lax.axis_index("c")
        base = wid * bpw
        for i in range(nit):
            off = base + i * chunk
            pltpu.sync_copy(idx_hbm.at[pl.ds(off, chunk)], idx_v)
            pltpu.async_copy(table_hbm.at[idx_v], rows_v, sem).wait()
            pltpu.sync_copy(rows_v, out_hbm.at[pl.ds(off, chunk)])

    return k(table, idx)


def _knn_body(dst_ref, src_ref, out_ref, *, ns):
    dst = dst_ref[0]
    src = src_ref[0]
    d = None
    for c in range(3):
        diff = dst[:, c:c + 1] - src[c:c + 1, :]
        d = diff * diff if d is None else d + diff * diff
    cols = jax.lax.broadcasted_iota(jnp.int32, d.shape, 1)
    idxs = []
    big = jnp.float32(jnp.inf)
    for _ in range(_K):
        m = jnp.min(d, axis=1, keepdims=True)
        cand = jnp.where(d == m, cols, jnp.int32(ns))
        idx = jnp.min(cand, axis=1, keepdims=True)
        idxs.append(idx)
        d = jnp.where(cols == idx, big, d)
    out_ref[0] = jnp.concatenate(idxs, axis=1)


def _knn_call(dst_p, src_t, tr):
    b, nd, _ = dst_p.shape
    ns = src_t.shape[2]
    grid = (b, nd // tr)
    return pl.pallas_call(
        functools.partial(_knn_body, ns=ns),
        grid=grid,
        in_specs=[
            pl.BlockSpec((1, tr, 4), lambda bb, i: (bb, i, 0)),
            pl.BlockSpec((1, 4, ns), lambda bb, i: (bb, 0, 0)),
        ],
        out_specs=pl.BlockSpec((1, tr, _K), lambda bb, i: (bb, i, 0)),
        out_shape=jax.ShapeDtypeStruct((b, nd, _K), jnp.int32),
    )(dst_p, src_t)


def _ptconv_body(pts_ref, dst_ref, feats_ref, cflat_ref, l1_ref, b1_ref,
                 l2_ref, b2_ref, l3_ref, b3_ref, w2_ref, out_ref):
    dstp = dst_ref[0]
    rels = []
    maxsq = None
    for k in range(_K):
        rel = pts_ref[0, k] - dstp
        sq = jnp.sum(rel * rel, axis=1, keepdims=True)
        maxsq = sq if maxsq is None else jnp.maximum(maxsq, sq)
        rels.append(rel)
    s = jnp.sqrt(maxsq)
    inv = 1.0 / jnp.where(s == 0.0, 1.0, s)

    cflat = cflat_ref[0]
    l1 = l1_ref[...]
    l2 = l2_ref[...]
    l3 = l3_ref[...]
    b1 = b1_ref[...]
    b2 = b2_ref[...]
    b3 = b3_ref[...]

    cin = feats_ref.shape[3]
    tt = dstp.shape[0]
    fs = [jnp.zeros((tt, cin), jnp.float32) for _ in range(_NC)]
    for k in range(_K):
        r3 = (rels[k] * inv)[:, 0:3]
        d48 = jnp.concatenate([r3] * _NC, axis=1) - cflat[None, :]
        h = jnp.maximum(jnp.dot(d48, l1) + b1, 0.0)
        h = jnp.maximum(jnp.dot(h, l2) + b2, 0.0)
        h = jnp.dot(h, l3) + b3
        fk = feats_ref[0, k]
        for m in range(_NC):
            fs[m] = fs[m] + fk * h[:, m:m + 1]
    acc = None
    for m in range(_NC):
        part = jnp.dot(fs[m], w2_ref[m])
        acc = part if acc is None else acc + part
    out_ref[0] = acc * (1.0 / float(_K * _NC))


def _ptconv_call(pts_g, dst_p, feats_g, p, cout, tile):
    b, _, nd, _ = pts_g.shape
    cin = feats_g.shape[3]
    cflat = p["centers"].reshape(1, 3 * _NC)
    true_cin = p["W"].shape[0] // _NC
    w2 = p["W"].reshape(true_cin, _NC, cout)
    if true_cin != cin:
        w2 = jnp.pad(w2, ((0, cin - true_cin), (0, 0), (0, 0)))
    w2 = jnp.transpose(w2, (1, 0, 2))
    grid = (b, nd // tile)
    return pl.pallas_call(
        _ptconv_body,
        grid=grid,
        in_specs=[
            pl.BlockSpec((1, _K, tile, 4), lambda bb, i: (bb, 0, i, 0)),
            pl.BlockSpec((1, tile, 4), lambda bb, i: (bb, i, 0)),
            pl.BlockSpec((1, _K, tile, cin), lambda bb, i: (bb, 0, i, 0)),
            pl.BlockSpec((1, 3 * _NC), lambda bb, i: (0, 0)),
            pl.BlockSpec((3 * _NC, 2 * _NC), lambda bb, i: (0, 0)),
            pl.BlockSpec((1, 2 * _NC), lambda bb, i: (0, 0)),
            pl.BlockSpec((2 * _NC, _NC), lambda bb, i: (0, 0)),
            pl.BlockSpec((1, _NC), lambda bb, i: (0, 0)),
            pl.BlockSpec((_NC, _NC), lambda bb, i: (0, 0)),
            pl.BlockSpec((1, _NC), lambda bb, i: (0, 0)),
            pl.BlockSpec((_NC, cin, cout), lambda bb, i: (0, 0, 0)),
        ],
        out_specs=pl.BlockSpec((1, tile, cout), lambda bb, i: (bb, i, 0)),
        out_shape=jax.ShapeDtypeStruct((b, nd, cout), jnp.float32),
    )(pts_g, dst_p, feats_g, cflat, p["l1_w"], p["l1_b"].reshape(1, -1),
      p["l2_w"], p["l2_b"].reshape(1, -1), p["l3_w"], p["l3_b"].reshape(1, -1),
      w2)


def _bn_relu(p, x):
    mean = jnp.mean(x, axis=(0, 1))
    var = jnp.var(x, axis=(0, 1))
    xh = (x - mean) / jnp.sqrt(var + 1e-5)
    return jax.nn.relu(xh * p["bn_gamma"] + p["bn_beta"])


def _level(p, feats, pos_pad, pos16_flat, ns, nd, ids, cout, tile):
    b = feats.shape[0]
    n_all = pos_pad.shape[1]
    ids_t = jnp.transpose(ids, (0, 2, 1))
    cin = feats.shape[2]
    if cin == 3:
        feats = jnp.pad(feats, ((0, 0), (0, 0), (0, 13)))
        cin = 16
    boff = jnp.arange(b, dtype=jnp.int32)[:, None, None]
    idx_pts = (ids_t + boff * n_all).reshape(-1)
    idx_f = (ids_t + boff * feats.shape[1]).reshape(-1)
    pts_g = _sc_gather(pos16_flat, idx_pts).reshape(b, _K, nd, 16)[..., :4]
    feats_g = _sc_gather(feats.reshape(-1, cin), idx_f).reshape(b, _K, nd, cin)
    dst_p = pos_pad[:, :nd]
    out = _ptconv_call(pts_g, dst_p, feats_g, p, cout, tile)
    return _bn_relu(p, out)


def kernel(x, pos, params):
    b, _, n = x.shape
    xf = jnp.transpose(x, (0, 2, 1))
    pos_pad = jnp.pad(pos, ((0, 0), (0, 0), (0, 1)))
    pos_t = jnp.transpose(pos, (0, 2, 1))
    pos_t = jnp.pad(pos_t, ((0, 0), (0, 1), (0, 0)))

    ids1 = _knn_call(pos_pad, pos_t, 128)
    ids2 = ids1[:, :512]
    ids3 = _knn_call(pos_pad[:, :128], pos_t[:, :, :512], 128)
    ids4 = _knn_call(pos_pad[:, :32], pos_t[:, :, :128], 32)
    ids5 = _knn_call(pos_pad[:, :8], pos_t[:, :, :32], 8)

    pos16 = jnp.pad(pos, ((0, 0), (0, 0), (0, 13))).reshape(-1, 16)

    h = _level(params["c1"], xf, pos_pad, pos16, 2048, 2048, ids1, 64, 128)
    h = _level(params["c2"], h, pos_pad, pos16, 2048, 512, ids2, 128, 128)
    h = _level(params["c3"], h, pos_pad, pos16, 512, 128, ids3, 256, 128)
    h = _level(params["c4"], h, pos_pad, pos16, 128, 32, ids4, 256, 32)
    h = _level(params["c5"], h, pos_pad, pos16, 32, 8, ids5, 512, 8)

    g = jnp.mean(h, axis=1)
    return g @ params["fc_w"] + params["fc_b"]


# speedup vs baseline: 1.0546x; 1.0546x over previous
import functools
import jax
import jax.numpy as jnp
from jax import lax
from jax.experimental import pallas as pl
from jax.experimental.pallas import tpu as pltpu
from jax.experimental.pallas import tpu_sc as plsc

_NC = 16
_K = 16


def _sc_gather(table, idx):
    r, d = table.shape
    m = idx.shape[0]
    nw = 32
    bpw = m // nw
    chunk = min(bpw, max(8, (64 * 1024) // (4 * d)))
    nit = bpw // chunk
    mesh = plsc.VectorSubcoreMesh(core_axis_name="c", subcore_axis_name="s")

    @functools.partial(
        pl.kernel, mesh=mesh,
        compiler_params=pltpu.CompilerParams(use_tc_tiling_on_sc=False),
        out_type=jax.ShapeDtypeStruct((m, d), jnp.float32),
        scratch_types=[
            pltpu.VMEM((chunk,), jnp.int32),
            pltpu.VMEM((chunk, d), jnp.float32),
            pltpu.SemaphoreType.DMA,
        ],
    )
    def k(table_hbm, idx_hbm, out_hbm, idx_v, rows_v, sem):
        wid = lax.axis_index("s") * 2 + lax.axis_index("c")
        base = wid * bpw
        for i in range(nit):
            off = base + i * chunk
            pltpu.sync_copy(idx_hbm.at[pl.ds(off, chunk)], idx_v)
            pltpu.async_copy(table_hbm.at[idx_v], rows_v, sem).wait()
            pltpu.sync_copy(rows_v, out_hbm.at[pl.ds(off, chunk)])

    return k(table, idx)


def _knn_body(dst_ref, src_ref, out_ref, *, ns):
    dst = dst_ref[0]
    src = src_ref[0]
    d = None
    for c in range(3):
        diff = dst[:, c:c + 1] - src[c:c + 1, :]
        d = diff * diff if d is None else d + diff * diff
    del ns
    cols = jax.lax.broadcasted_iota(jnp.int32, d.shape, 1)
    di = pltpu.bitcast(d, jnp.int32)
    packed = jnp.bitwise_or(jnp.bitwise_and(di, jnp.int32(~2047)), cols)
    idxs = []
    for _ in range(_K):
        m = jnp.min(packed, axis=1, keepdims=True)
        idxs.append(jnp.bitwise_and(m, jnp.int32(2047)))
        packed = jnp.where(packed == m, jnp.int32(0x7FFFFFFF), packed)
    out_ref[0] = jnp.concatenate(idxs, axis=1)


def _knn_call(dst_p, src_t, tr):
    b, nd, _ = dst_p.shape
    ns = src_t.shape[2]
    grid = (b, nd // tr)
    return pl.pallas_call(
        functools.partial(_knn_body, ns=ns),
        grid=grid,
        in_specs=[
            pl.BlockSpec((1, tr, 4), lambda bb, i: (bb, i, 0)),
            pl.BlockSpec((1, 4, ns), lambda bb, i: (bb, 0, 0)),
        ],
        out_specs=pl.BlockSpec((1, tr, _K), lambda bb, i: (bb, i, 0)),
        out_shape=jax.ShapeDtypeStruct((b, nd, _K), jnp.int32),
    )(dst_p, src_t)


def _ptconv_body(pts_ref, dst_ref, feats_ref, cflat_ref, l1_ref, b1_ref,
                 l2_ref, b2_ref, l3_ref, b3_ref, w2_ref, out_ref):
    dstp = dst_ref[0]
    rels = []
    maxsq = None
    for k in range(_K):
        rel = pts_ref[0, k] - dstp
        sq = jnp.sum(rel * rel, axis=1, keepdims=True)
        maxsq = sq if maxsq is None else jnp.maximum(maxsq, sq)
        rels.append(rel)
    s = jnp.sqrt(maxsq)
    inv = 1.0 / jnp.where(s == 0.0, 1.0, s)

    cflat = cflat_ref[0]
    l1 = l1_ref[...]
    l2 = l2_ref[...]
    l3 = l3_ref[...]
    b1 = b1_ref[...]
    b2 = b2_ref[...]
    b3 = b3_ref[...]

    cin = feats_ref.shape[3]
    tt = dstp.shape[0]
    fs = [jnp.zeros((tt, cin), jnp.float32) for _ in range(_NC)]
    for k in range(_K):
        r3 = (rels[k] * inv)[:, 0:3]
        d48 = jnp.concatenate([r3] * _NC, axis=1) - cflat[None, :]
        h = jnp.maximum(jnp.dot(d48, l1) + b1, 0.0)
        h = jnp.maximum(jnp.dot(h, l2) + b2, 0.0)
        h = jnp.dot(h, l3) + b3
        fk = feats_ref[0, k]
        for m in range(_NC):
            fs[m] = fs[m] + fk * h[:, m:m + 1]
    acc = None
    for m in range(_NC):
        part = jnp.dot(fs[m], w2_ref[m])
        acc = part if acc is None else acc + part
    out_ref[0] = acc * (1.0 / float(_K * _NC))


def _ptconv_call(pts_g, dst_p, feats_g, p, cout, tile):
    b, _, nd, _ = pts_g.shape
    cin = feats_g.shape[3]
    cflat = p["centers"].reshape(1, 3 * _NC)
    true_cin = p["W"].shape[0] // _NC
    w2 = p["W"].reshape(true_cin, _NC, cout)
    if true_cin != cin:
        w2 = jnp.pad(w2, ((0, cin - true_cin), (0, 0), (0, 0)))
    w2 = jnp.transpose(w2, (1, 0, 2))
    grid = (b, nd // tile)
    return pl.pallas_call(
        _ptconv_body,
        grid=grid,
        in_specs=[
            pl.BlockSpec((1, _K, tile, 4), lambda bb, i: (bb, 0, i, 0)),
            pl.BlockSpec((1, tile, 4), lambda bb, i: (bb, i, 0)),
            pl.BlockSpec((1, _K, tile, cin), lambda bb, i: (bb, 0, i, 0)),
            pl.BlockSpec((1, 3 * _NC), lambda bb, i: (0, 0)),
            pl.BlockSpec((3 * _NC, 2 * _NC), lambda bb, i: (0, 0)),
            pl.BlockSpec((1, 2 * _NC), lambda bb, i: (0, 0)),
            pl.BlockSpec((2 * _NC, _NC), lambda bb, i: (0, 0)),
            pl.BlockSpec((1, _NC), lambda bb, i: (0, 0)),
            pl.BlockSpec((_NC, _NC), lambda bb, i: (0, 0)),
            pl.BlockSpec((1, _NC), lambda bb, i: (0, 0)),
            pl.BlockSpec((_NC, cin, cout), lambda bb, i: (0, 0, 0)),
        ],
        out_specs=pl.BlockSpec((1, tile, cout), lambda bb, i: (bb, i, 0)),
        out_shape=jax.ShapeDtypeStruct((b, nd, cout), jnp.float32),
    )(pts_g, dst_p, feats_g, cflat, p["l1_w"], p["l1_b"].reshape(1, -1),
      p["l2_w"], p["l2_b"].reshape(1, -1), p["l3_w"], p["l3_b"].reshape(1, -1),
      w2)


def _bn_relu(p, x):
    mean = jnp.mean(x, axis=(0, 1))
    var = jnp.var(x, axis=(0, 1))
    xh = (x - mean) / jnp.sqrt(var + 1e-5)
    return jax.nn.relu(xh * p["bn_gamma"] + p["bn_beta"])


def _level(p, feats, pos_pad, pos16_flat, ns, nd, ids, cout, tile):
    b = feats.shape[0]
    n_all = pos_pad.shape[1]
    ids_t = jnp.transpose(ids, (0, 2, 1))
    cin = feats.shape[2]
    if cin == 3:
        feats = jnp.pad(feats, ((0, 0), (0, 0), (0, 13)))
        cin = 16
    boff = jnp.arange(b, dtype=jnp.int32)[:, None, None]
    idx_pts = (ids_t + boff * n_all).reshape(-1)
    idx_f = (ids_t + boff * feats.shape[1]).reshape(-1)
    pts_g = _sc_gather(pos16_flat, idx_pts).reshape(b, _K, nd, 16)[..., :4]
    feats_g = _sc_gather(feats.reshape(-1, cin), idx_f).reshape(b, _K, nd, cin)
    dst_p = pos_pad[:, :nd]
    out = _ptconv_call(pts_g, dst_p, feats_g, p, cout, tile)
    return _bn_relu(p, out)


def kernel(x, pos, params):
    b, _, n = x.shape
    xf = jnp.transpose(x, (0, 2, 1))
    pos_pad = jnp.pad(pos, ((0, 0), (0, 0), (0, 1)))
    pos_t = jnp.transpose(pos, (0, 2, 1))
    pos_t = jnp.pad(pos_t, ((0, 0), (0, 1), (0, 0)))

    ids1 = _knn_call(pos_pad, pos_t, 128)
    ids2 = ids1[:, :512]
    ids3 = _knn_call(pos_pad[:, :128], pos_t[:, :, :512], 128)
    ids4 = _knn_call(pos_pad[:, :32], pos_t[:, :, :128], 32)
    ids5 = _knn_call(pos_pad[:, :8], pos_t[:, :, :32], 8)

    pos16 = jnp.pad(pos, ((0, 0), (0, 0), (0, 13))).reshape(-1, 16)

    h = _level(params["c1"], xf, pos_pad, pos16, 2048, 2048, ids1, 64, 128)
    h = _level(params["c2"], h, pos_pad, pos16, 2048, 512, ids2, 128, 128)
    h = _level(params["c3"], h, pos_pad, pos16, 512, 128, ids3, 256, 128)
    h = _level(params["c4"], h, pos_pad, pos16, 128, 32, ids4, 256, 32)
    h = _level(params["c5"], h, pos_pad, pos16, 32, 8, ids5, 512, 8)

    g = jnp.mean(h, axis=1)
    return g @ params["fc_w"] + params["fc_b"]


# speedup vs baseline: 1.1723x; 1.1116x over previous
import functools
import jax
import jax.numpy as jnp
from jax import lax
from jax.experimental import pallas as pl
from jax.experimental.pallas import tpu as pltpu
from jax.experimental.pallas import tpu_sc as plsc

_NC = 16
_K = 16


def _sc_gather(table, idx):
    r, d = table.shape
    m = idx.shape[0]
    nw = 32
    bpw = m // nw
    chunk = min(bpw, max(8, (64 * 1024) // (4 * d)))
    nit = bpw // chunk
    mesh = plsc.VectorSubcoreMesh(core_axis_name="c", subcore_axis_name="s")

    @functools.partial(
        pl.kernel, mesh=mesh,
        compiler_params=pltpu.CompilerParams(use_tc_tiling_on_sc=False),
        out_type=jax.ShapeDtypeStruct((m, d), jnp.float32),
        scratch_types=[
            pltpu.VMEM((chunk,), jnp.int32),
            pltpu.VMEM((chunk, d), jnp.float32),
            pltpu.SemaphoreType.DMA,
        ],
    )
    def k(table_hbm, idx_hbm, out_hbm, idx_v, rows_v, sem):
        wid = lax.axis_index("s") * 2 + lax.axis_index("c")
        base = wid * bpw
        for i in range(nit):
            off = base + i * chunk
            pltpu.sync_copy(idx_hbm.at[pl.ds(off, chunk)], idx_v)
            pltpu.async_copy(table_hbm.at[idx_v], rows_v, sem).wait()
            pltpu.sync_copy(rows_v, out_hbm.at[pl.ds(off, chunk)])

    return k(table, idx)


def _knn_body(dst_ref, src_ref, out_ref, *, ns):
    dst = dst_ref[0]
    src = src_ref[0]
    d = None
    for c in range(3):
        diff = dst[:, c:c + 1] - src[c:c + 1, :]
        d = diff * diff if d is None else d + diff * diff
    del ns
    cols = jax.lax.broadcasted_iota(jnp.int32, d.shape, 1)
    di = pltpu.bitcast(d, jnp.int32)
    packed = jnp.bitwise_or(jnp.bitwise_and(di, jnp.int32(~2047)), cols)
    idxs = []
    for _ in range(_K):
        m = jnp.min(packed, axis=1, keepdims=True)
        idxs.append(jnp.bitwise_and(m, jnp.int32(2047)))
        packed = jnp.where(packed == m, jnp.int32(0x7FFFFFFF), packed)
    out_ref[0] = jnp.concatenate(idxs, axis=1)


def _knn_call(dst_p, src_t, tr):
    b, nd, _ = dst_p.shape
    ns = src_t.shape[2]
    grid = (b, nd // tr)
    return pl.pallas_call(
        functools.partial(_knn_body, ns=ns),
        grid=grid,
        in_specs=[
            pl.BlockSpec((1, tr, 4), lambda bb, i: (bb, i, 0)),
            pl.BlockSpec((1, 4, ns), lambda bb, i: (bb, 0, 0)),
        ],
        out_specs=pl.BlockSpec((1, tr, _K), lambda bb, i: (bb, i, 0)),
        out_shape=jax.ShapeDtypeStruct((b, nd, _K), jnp.int32),
    )(dst_p, src_t)


def _ptconv_body(pts_ref, dst_ref, feats_ref, cflat_ref, l1_ref, b1_ref,
                 l2_ref, b2_ref, l3_ref, b3_ref, w2_ref, out_ref):
    dstp = dst_ref[0]
    rels = []
    maxsq = None
    for k in range(_K):
        rel = pts_ref[0, k] - dstp
        sq = jnp.sum(rel * rel, axis=1, keepdims=True)
        maxsq = sq if maxsq is None else jnp.maximum(maxsq, sq)
        rels.append(rel)
    s = jnp.sqrt(maxsq)
    inv = 1.0 / jnp.where(s == 0.0, 1.0, s)

    cflat = cflat_ref[0]
    l1 = l1_ref[...]
    l2 = l2_ref[...]
    l3 = l3_ref[...]
    b1 = b1_ref[...]
    b2 = b2_ref[...]
    b3 = b3_ref[...]

    cin = feats_ref.shape[3]
    tt = dstp.shape[0]
    fs = [jnp.zeros((tt, cin), jnp.float32) for _ in range(_NC)]
    for k in range(_K):
        r3 = (rels[k] * inv)[:, 0:3]
        d48 = jnp.concatenate([r3] * _NC, axis=1) - cflat[None, :]
        h = jnp.maximum(jnp.dot(d48, l1) + b1, 0.0)
        h = jnp.maximum(jnp.dot(h, l2) + b2, 0.0)
        h = jnp.dot(h, l3) + b3
        fk = feats_ref[0, k]
        for m in range(_NC):
            fs[m] = fs[m] + fk * h[:, m:m + 1]
    acc = None
    for m in range(_NC):
        part = jnp.dot(fs[m], w2_ref[m])
        acc = part if acc is None else acc + part
    out_ref[0] = acc * (1.0 / float(_K * _NC))


def _ptconv_call(pts_g, dst_p, feats_g, p, cout, tile):
    b, _, nd, _ = pts_g.shape
    cin = feats_g.shape[3]
    cflat = p["centers"].reshape(1, 3 * _NC)
    true_cin = p["W"].shape[0] // _NC
    w2 = p["W"].reshape(true_cin, _NC, cout)
    if true_cin != cin:
        w2 = jnp.pad(w2, ((0, cin - true_cin), (0, 0), (0, 0)))
    w2 = jnp.transpose(w2, (1, 0, 2))
    grid = (b, nd // tile)
    return pl.pallas_call(
        _ptconv_body,
        grid=grid,
        in_specs=[
            pl.BlockSpec((1, _K, tile, 4), lambda bb, i: (bb, 0, i, 0)),
            pl.BlockSpec((1, tile, 4), lambda bb, i: (bb, i, 0)),
            pl.BlockSpec((1, _K, tile, cin), lambda bb, i: (bb, 0, i, 0)),
            pl.BlockSpec((1, 3 * _NC), lambda bb, i: (0, 0)),
            pl.BlockSpec((3 * _NC, 2 * _NC), lambda bb, i: (0, 0)),
            pl.BlockSpec((1, 2 * _NC), lambda bb, i: (0, 0)),
            pl.BlockSpec((2 * _NC, _NC), lambda bb, i: (0, 0)),
            pl.BlockSpec((1, _NC), lambda bb, i: (0, 0)),
            pl.BlockSpec((_NC, _NC), lambda bb, i: (0, 0)),
            pl.BlockSpec((1, _NC), lambda bb, i: (0, 0)),
            pl.BlockSpec((_NC, cin, cout), lambda bb, i: (0, 0, 0)),
        ],
        out_specs=pl.BlockSpec((1, tile, cout), lambda bb, i: (bb, i, 0)),
        out_shape=jax.ShapeDtypeStruct((b, nd, cout), jnp.float32),
    )(pts_g, dst_p, feats_g, cflat, p["l1_w"], p["l1_b"].reshape(1, -1),
      p["l2_w"], p["l2_b"].reshape(1, -1), p["l3_w"], p["l3_b"].reshape(1, -1),
      w2)


def _bn_relu(p, x):
    mean = jnp.mean(x, axis=(0, 1))
    var = jnp.var(x, axis=(0, 1))
    xh = (x - mean) / jnp.sqrt(var + 1e-5)
    return jax.nn.relu(xh * p["bn_gamma"] + p["bn_beta"])


def _level(p, feats, pos_pad, pos16_flat, ns, nd, ids, cout, tile):
    b = feats.shape[0]
    n_all = pos_pad.shape[1]
    ids_t = jnp.transpose(ids, (0, 2, 1))
    cin = feats.shape[2]
    if cin == 3:
        feats = jnp.pad(feats, ((0, 0), (0, 0), (0, 13)))
        cin = 16
    boff = jnp.arange(b, dtype=jnp.int32)[:, None, None]
    idx_pts = (ids_t + boff * n_all).reshape(-1)
    idx_f = (ids_t + boff * feats.shape[1]).reshape(-1)
    pts_g = _sc_gather(pos16_flat, idx_pts).reshape(b, _K, nd, 16)[..., :4]
    feats_g = _sc_gather(feats.reshape(-1, cin), idx_f).reshape(b, _K, nd, cin)
    dst_p = pos_pad[:, :nd]
    out = _ptconv_call(pts_g, dst_p, feats_g, p, cout, tile)
    return _bn_relu(p, out)


def kernel(x, pos, params):
    b, _, n = x.shape
    xf = jnp.transpose(x, (0, 2, 1))
    pos_pad = jnp.pad(pos, ((0, 0), (0, 0), (0, 1)))
    pos_t = jnp.transpose(pos, (0, 2, 1))
    pos_t = jnp.pad(pos_t, ((0, 0), (0, 1), (0, 0)))

    ids1 = _knn_call(pos_pad, pos_t, 256)
    ids2 = ids1[:, :512]
    ids3 = _knn_call(pos_pad[:, :128], pos_t[:, :, :512], 128)
    ids4 = _knn_call(pos_pad[:, :32], pos_t[:, :, :128], 32)
    ids5 = _knn_call(pos_pad[:, :8], pos_t[:, :, :32], 8)

    pos16 = jnp.pad(pos, ((0, 0), (0, 0), (0, 13))).reshape(-1, 16)

    h = _level(params["c1"], xf, pos_pad, pos16, 2048, 2048, ids1, 64, 512)
    h = _level(params["c2"], h, pos_pad, pos16, 2048, 512, ids2, 128, 512)
    h = _level(params["c3"], h, pos_pad, pos16, 512, 128, ids3, 256, 128)
    h = _level(params["c4"], h, pos_pad, pos16, 128, 32, ids4, 256, 32)
    h = _level(params["c5"], h, pos_pad, pos16, 32, 8, ids5, 512, 8)

    g = jnp.mean(h, axis=1)
    return g @ params["fc_w"] + params["fc_b"]
